# baseline (speedup 1.0000x reference)
import functools

import jax
import jax.numpy as jnp
from jax import lax
from jax.experimental import pallas as pl
from jax.experimental.pallas import tpu as pltpu
from jax.experimental.pallas import tpu_sc as plsc

SEQ_LEN = 200
HID = 128
TAGS = 1000
ROWS_PER_W = 16
N_WORKERS = 13
N_ROWS_OUT = N_WORKERS * ROWS_PER_W


def _sc_gather(sentence, emb_table):
    mesh = plsc.VectorSubcoreMesh(core_axis_name="c", subcore_axis_name="s",
                                  num_cores=1)

    @functools.partial(
        pl.kernel,
        mesh=mesh,
        out_type=jax.ShapeDtypeStruct((N_ROWS_OUT, HID), jnp.float32),
        compiler_params=pltpu.CompilerParams(
            disable_bounds_checks=True,
            disable_semaphore_checks=True,
            skip_device_barrier=True,
        ),
        scratch_types=[
            pltpu.VMEM((ROWS_PER_W,), jnp.int32),
            pltpu.VMEM((ROWS_PER_W, HID), jnp.float32),
            pltpu.SemaphoreType.DMA,
        ],
    )
    def k(sent_hbm, table_hbm, out_hbm, idx_v, rows_v, sem):
        wid = lax.axis_index("s")

        @pl.when(wid < N_WORKERS)
        def _():
            base = wid * ROWS_PER_W - jnp.where(wid == N_WORKERS - 1, 8, 0)
            pltpu.sync_copy(sent_hbm.at[pl.ds(base, ROWS_PER_W)], idx_v)
            pltpu.async_copy(table_hbm.at[idx_v], rows_v, sem).wait()
            pltpu.sync_copy(rows_v,
                            out_hbm.at[pl.ds(wid * ROWS_PER_W, ROWS_PER_W)])

    return k(sentence, emb_table)


def _tc_head(rows, W, b2):
    def body(p_ref, w_ref, b_ref, o_ref):
        psum = (jnp.sum(p_ref[0:192, :], axis=0, keepdims=True)
                + jnp.sum(p_ref[200:208, :], axis=0, keepdims=True))
        vec = psum * (1.0 / SEQ_LEN)
        tag = lax.dot_general(vec, w_ref[...], (((1,), (1,)), ((), ())),
                              preferred_element_type=jnp.float32)
        tag = tag + b_ref[...]
        m = jnp.max(tag, axis=1, keepdims=True)
        e = jnp.exp(tag - m)
        s = jnp.sum(e, axis=1, keepdims=True)
        o_ref[...] = tag - m - jnp.log(s)

    return pl.pallas_call(
        body,
        out_shape=jax.ShapeDtypeStruct((1, TAGS), jnp.float32),
    )(rows, W, b2)


def kernel(sentence, emb_table, W, b):
    sentence = sentence.astype(jnp.int32)
    rows = _sc_gather(sentence, emb_table)
    return _tc_head(rows, W, b.reshape(1, TAGS))
